# baseline (speedup 1.0000x reference)
import jax
import jax.numpy as jnp
from jax import lax
from jax.experimental import pallas as pl
from jax.experimental.pallas import tpu as pltpu
from jax.experimental.pallas import tpu_sc as plsc

B, Q, T, C = 128, 900, 256, 91
CPAD = 128
QPAD = 1024
K = 300
BBLK = 4
NB = B // BBLK

NC, NS, L = 2, 16, 16
NW = NC * NS
B_PER_W = B // NW

NCHUNK = 8
CROWS = QPAD // NCHUNK
CANDBUF = 2064
KPAD = 304
HI_BITS = 0x43800000


def _tc_body(logits_ref, pmap_ref, prob_ref, t_ref):
    x = logits_ref[...]
    s = jax.nn.sigmoid(x).reshape(BBLK * Q, T)
    pm = pmap_ref[...]
    prob = lax.dot_general(
        s, pm, dimension_numbers=(((1,), (1,)), ((), ())),
        preferred_element_type=jnp.float32)
    probp = jnp.concatenate(
        [prob, jnp.full((BBLK * Q, CPAD - C), -1.0, jnp.float32)], axis=1)
    probp = probp.reshape(BBLK, Q, CPAD)
    prob_ref[...] = jnp.concatenate(
        [probp, jnp.full((BBLK, QPAD - Q, CPAD), -1.0, jnp.float32)], axis=1)

    bits = lax.bitcast_convert_type(probp, jnp.int32)
    lo = jnp.zeros((BBLK, 1, 1), jnp.int32)
    hi = jnp.full((BBLK, 1, 1), (HI_BITS >> 16) + 1, jnp.int32)

    def count_ge(m):
        return jnp.sum((bits >= (m << 16)).astype(jnp.int32), axis=(1, 2),
                       keepdims=True)

    def step(_, lohi):
        lo, hi = lohi
        w = hi - lo
        m1 = lo + (w >> 2)
        m2 = lo + (w >> 1)
        m3 = hi - (w >> 2)
        c1 = count_ge(m1) >= K
        c2 = count_ge(m2) >= K
        c3 = count_ge(m3) >= K
        new_lo = jnp.where(c3, m3, jnp.where(c2, m2, jnp.where(c1, m1, lo)))
        new_hi = jnp.where(c3, hi, jnp.where(c2, m3, jnp.where(c1, m2, m1)))
        return new_lo, new_hi

    lo, hi = lax.fori_loop(0, 8, step, (lo, hi))
    t = lax.bitcast_convert_type(lo << 16, jnp.float32).reshape(1, BBLK, 1)
    t_ref[...] = jnp.broadcast_to(t, (1, BBLK, L))


def _tc_stage(pred_logits, positive_map):
    return pl.pallas_call(
        _tc_body,
        grid=(NB,),
        in_specs=[
            pl.BlockSpec((BBLK, Q, T), lambda i: (i, 0, 0)),
            pl.BlockSpec((C, T), lambda i: (0, 0)),
        ],
        out_specs=[
            pl.BlockSpec((BBLK, QPAD, CPAD), lambda i: (i, 0, 0)),
            pl.BlockSpec((1, BBLK, L), lambda i: (i, 0, 0)),
        ],
        out_shape=[
            jax.ShapeDtypeStruct((B, QPAD, CPAD), jnp.float32),
            jax.ShapeDtypeStruct((NB, BBLK, L), jnp.float32),
        ],
    )(pred_logits, positive_map)


def _sc_body(prob_hbm, aux_hbm, boxes_hbm,
             scores_hbm, labels_hbm, boxout_hbm,
             chunk_v, val_v, idx_v, box_v, aux_v, sc_v, lb_v, bx_v):
    wid = lax.axis_index("s") * NC + lax.axis_index("c")
    iota = lax.iota(jnp.int32, L)

    for i in range(B_PER_W):
        b = wid * B_PER_W + i
        pltpu.sync_copy(aux_hbm.at[pl.ds(b * 3 * L, 3 * L)], aux_v)
        pltpu.sync_copy(boxes_hbm.at[pl.ds(b * Q * 4, Q * 4)], box_v)
        t_vec = aux_v[0:L]
        w_vec = aux_v[L:2 * L]
        h_vec = aux_v[2 * L:3 * L]

        iota_k = [iota + (rr * CPAD + c8 * L)
                  for rr in range(2) for c8 in range(CPAD // L)]

        def chunk_loop(ci, cnt):
            pltpu.sync_copy(prob_hbm.at[b, pl.ds(ci * CROWS, CROWS), :],
                            chunk_v)

            def blk(i2, cnt):
                r = i2 * 2
                vs = []
                ms = []
                for rr in range(2):
                    for c8 in range(CPAD // L):
                        v = chunk_v[r + rr, pl.ds(c8 * L, L)]
                        vs.append(v)
                        ms.append(v >= t_vec)
                anym = ms[0]
                for m in ms[1:]:
                    anym = anym | m
                hit = jnp.any(anym)

                def slow(cnt):
                    rowbase = (ci * CROWS + r) * CPAD
                    for k in range(2 * (CPAD // L)):
                        m = ms[k]
                        ok = cnt < CANDBUF - 2 * L
                        cc = jnp.where(ok, cnt, CANDBUF - 2 * L)
                        plsc.store_compressed(val_v.at[pl.ds(cc, L)],
                                              vs[k], mask=m)
                        plsc.store_compressed(idx_v.at[pl.ds(cc, L)],
                                              iota_k[k] + rowbase, mask=m)
                        n = plsc.all_reduce_population_count(m)[0]
                        cnt = cnt + jnp.where(ok, n, 0)
                    return cnt

                return lax.cond(hit, slow, lambda c: c, cnt)

            return lax.fori_loop(0, CROWS // 2, blk, cnt)

        cnt = lax.fori_loop(0, NCHUNK, chunk_loop, jnp.int32(0))
        val_v[pl.ds(cnt, L)] = jnp.full((L,), -1.0, jnp.float32)

        ngroups = (cnt + L - 1) >> 4

        def rank_group(g, _):
            vq = val_v[pl.ds(g * L, L)]
            iq = idx_v[pl.ds(g * L, L)]

            def eblk(eg, rank):
                vb16 = val_v[pl.ds(eg * L, L)]
                ib16 = idx_v[pl.ds(eg * L, L)]
                for j in range(L):
                    vb = vb16[j]
                    ib = ib16[j]
                    beat = (vb > vq) | ((vb == vq) & (ib < iq))
                    rank = rank + beat.astype(jnp.int32)
                return rank

            rank = lax.fori_loop(0, ngroups, eblk,
                                 jnp.zeros((L,), jnp.int32))
            sel = rank < K
            plsc.store_scatter(sc_v, [rank], vq, mask=sel)
            c_idx = iq & (CPAD - 1)
            q_idx = iq >> 7
            plsc.store_scatter(lb_v, [rank], c_idx, mask=sel)
            q4 = q_idx * 4
            cx = plsc.load_gather(box_v, [q4], mask=sel)
            cy = plsc.load_gather(box_v, [q4 + 1], mask=sel)
            w = plsc.load_gather(box_v, [q4 + 2], mask=sel)
            h = plsc.load_gather(box_v, [q4 + 3], mask=sel)
            r4 = rank * 4
            plsc.store_scatter(bx_v, [r4], (cx - 0.5 * w) * w_vec, mask=sel)
            plsc.store_scatter(bx_v, [r4 + 1], (cy - 0.5 * h) * h_vec,
                               mask=sel)
            plsc.store_scatter(bx_v, [r4 + 2], (cx + 0.5 * w) * w_vec,
                               mask=sel)
            plsc.store_scatter(bx_v, [r4 + 3], (cy + 0.5 * h) * h_vec,
                               mask=sel)
            return 0

        lax.fori_loop(0, ngroups, rank_group, 0)

        pltpu.sync_copy(sc_v, scores_hbm.at[pl.ds(b * KPAD, KPAD)])
        pltpu.sync_copy(lb_v, labels_hbm.at[pl.ds(b * KPAD, KPAD)])
        pltpu.sync_copy(bx_v, boxout_hbm.at[pl.ds(b * 4 * KPAD, 4 * KPAD)])


_sc_stage = pl.kernel(
    _sc_body,
    out_type=[
        jax.ShapeDtypeStruct((B * KPAD,), jnp.float32),
        jax.ShapeDtypeStruct((B * KPAD,), jnp.int32),
        jax.ShapeDtypeStruct((B * 4 * KPAD,), jnp.float32),
    ],
    mesh=plsc.VectorSubcoreMesh(core_axis_name="c", subcore_axis_name="s",
                                num_cores=NC, num_subcores=NS),
    compiler_params=pltpu.CompilerParams(needs_layout_passes=False),
    scratch_types=[
        pltpu.VMEM((CROWS, CPAD), jnp.float32),
        pltpu.VMEM((CANDBUF,), jnp.float32),
        pltpu.VMEM((CANDBUF,), jnp.int32),
        pltpu.VMEM((Q * 4,), jnp.float32),
        pltpu.VMEM((3 * L,), jnp.float32),
        pltpu.VMEM((KPAD,), jnp.float32),
        pltpu.VMEM((KPAD,), jnp.int32),
        pltpu.VMEM((4 * KPAD,), jnp.float32),
    ],
)


def kernel(pred_logits, pred_boxes, target_sizes, positive_map):
    prob, tvals = _tc_stage(pred_logits, positive_map)
    tvals = tvals.reshape(B, L)
    img_h = jnp.broadcast_to(target_sizes[:, 0:1], (B, L))
    img_w = jnp.broadcast_to(target_sizes[:, 1:2], (B, L))
    aux = jnp.concatenate([tvals, img_w, img_h], axis=1).reshape(-1)
    boxes_flat = pred_boxes.reshape(-1)
    scores_p, labels_p, boxes_p = _sc_stage(prob, aux, boxes_flat)
    scores = scores_p.reshape(B, KPAD)[:, :K]
    labels = labels_p.reshape(B, KPAD)[:, :K]
    boxes = boxes_p.reshape(B, KPAD, 4)[:, :K, :]
    return scores, labels, boxes


# speedup vs baseline: 1.0419x; 1.0419x over previous
import jax
import jax.numpy as jnp
from jax import lax
from jax.experimental import pallas as pl
from jax.experimental.pallas import tpu as pltpu
from jax.experimental.pallas import tpu_sc as plsc

B, Q, T, C = 128, 900, 256, 91
CPAD = 128
QPAD = 1024
K = 300
BBLK = 16
NB = B // BBLK

NC, NS, L = 2, 16, 16
NW = NC * NS
B_PER_W = B // NW

NCHUNK = 8
CROWS = QPAD // NCHUNK
CANDBUF = 2064
KPAD = 304
HI_BITS = 0x43800000


def _tc_body(logits_ref, pmap_ref, prob_ref, t_ref):
    x = logits_ref[...]
    s = jax.nn.sigmoid(x).reshape(BBLK * Q, T)
    pm = pmap_ref[...]
    prob = lax.dot_general(
        s, pm, dimension_numbers=(((1,), (1,)), ((), ())),
        preferred_element_type=jnp.float32)
    probp = jnp.concatenate(
        [prob, jnp.full((BBLK * Q, CPAD - C), -1.0, jnp.float32)], axis=1)
    probp = probp.reshape(BBLK, Q, CPAD)
    prob_ref[...] = jnp.concatenate(
        [probp, jnp.full((BBLK, QPAD - Q, CPAD), -1.0, jnp.float32)], axis=1)

    bits = lax.bitcast_convert_type(probp, jnp.int32)
    lo = jnp.zeros((BBLK, 1, 1), jnp.int32)
    hi = jnp.full((BBLK, 1, 1), (HI_BITS >> 16) + 1, jnp.int32)

    def count_ge(m):
        return jnp.sum((bits >= (m << 16)).astype(jnp.int32), axis=(1, 2),
                       keepdims=True)

    def step(_, lohi):
        lo, hi = lohi
        w = hi - lo
        m1 = lo + (w >> 2)
        m2 = lo + (w >> 1)
        m3 = hi - (w >> 2)
        c1 = count_ge(m1) >= K
        c2 = count_ge(m2) >= K
        c3 = count_ge(m3) >= K
        new_lo = jnp.where(c3, m3, jnp.where(c2, m2, jnp.where(c1, m1, lo)))
        new_hi = jnp.where(c3, hi, jnp.where(c2, m3, jnp.where(c1, m2, m1)))
        return new_lo, new_hi

    lo, hi = lax.fori_loop(0, 8, step, (lo, hi))
    t = lax.bitcast_convert_type(lo << 16, jnp.float32).reshape(1, BBLK, 1)
    t_ref[...] = jnp.broadcast_to(t, (1, BBLK, L))


def _tc_stage(pred_logits, positive_map):
    return pl.pallas_call(
        _tc_body,
        grid=(NB,),
        in_specs=[
            pl.BlockSpec((BBLK, Q, T), lambda i: (i, 0, 0)),
            pl.BlockSpec((C, T), lambda i: (0, 0)),
        ],
        out_specs=[
            pl.BlockSpec((BBLK, QPAD, CPAD), lambda i: (i, 0, 0)),
            pl.BlockSpec((1, BBLK, L), lambda i: (i, 0, 0)),
        ],
        out_shape=[
            jax.ShapeDtypeStruct((B, QPAD, CPAD), jnp.float32),
            jax.ShapeDtypeStruct((NB, BBLK, L), jnp.float32),
        ],
    )(pred_logits, positive_map)


def _sc_body(prob_hbm, aux_hbm, boxes_hbm,
             scores_hbm, labels_hbm, boxout_hbm,
             chunk_v, val_v, idx_v, box_v, aux_v, sc_v, lb_v, bx_v):
    wid = lax.axis_index("s") * NC + lax.axis_index("c")
    iota = lax.iota(jnp.int32, L)

    for i in range(B_PER_W):
        b = wid * B_PER_W + i
        pltpu.sync_copy(aux_hbm.at[pl.ds(b * 3 * L, 3 * L)], aux_v)
        pltpu.sync_copy(boxes_hbm.at[pl.ds(b * Q * 4, Q * 4)], box_v)
        t_vec = aux_v[0:L]
        w_vec = aux_v[L:2 * L]
        h_vec = aux_v[2 * L:3 * L]

        iota_k = [iota + (rr * CPAD + c8 * L)
                  for rr in range(2) for c8 in range(CPAD // L)]

        def chunk_loop(ci, cnt):
            pltpu.sync_copy(prob_hbm.at[b, pl.ds(ci * CROWS, CROWS), :],
                            chunk_v)

            def blk(i2, cnt):
                r = i2 * 2
                vs = []
                ms = []
                for rr in range(2):
                    for c8 in range(CPAD // L):
                        v = chunk_v[r + rr, pl.ds(c8 * L, L)]
                        vs.append(v)
                        ms.append(v >= t_vec)
                anym = ms[0]
                for m in ms[1:]:
                    anym = anym | m
                hit = jnp.any(anym)

                def slow(cnt):
                    rowbase = (ci * CROWS + r) * CPAD
                    for k in range(2 * (CPAD // L)):
                        m = ms[k]
                        ok = cnt < CANDBUF - 2 * L
                        cc = jnp.where(ok, cnt, CANDBUF - 2 * L)
                        plsc.store_compressed(val_v.at[pl.ds(cc, L)],
                                              vs[k], mask=m)
                        plsc.store_compressed(idx_v.at[pl.ds(cc, L)],
                                              iota_k[k] + rowbase, mask=m)
                        n = plsc.all_reduce_population_count(m)[0]
                        cnt = cnt + jnp.where(ok, n, 0)
                    return cnt

                return lax.cond(hit, slow, lambda c: c, cnt)

            return lax.fori_loop(0, CROWS // 2, blk, cnt)

        cnt = lax.fori_loop(0, NCHUNK, chunk_loop, jnp.int32(0))
        val_v[pl.ds(cnt, L)] = jnp.full((L,), -1.0, jnp.float32)

        ngroups = (cnt + L - 1) >> 4

        def rank_group(g, _):
            vq = val_v[pl.ds(g * L, L)]
            iq = idx_v[pl.ds(g * L, L)]

            def eblk(eg, rank):
                vb16 = val_v[pl.ds(eg * L, L)]
                ib16 = idx_v[pl.ds(eg * L, L)]
                for j in range(L):
                    vb = vb16[j]
                    ib = ib16[j]
                    beat = (vb > vq) | ((vb == vq) & (ib < iq))
                    rank = rank + beat.astype(jnp.int32)
                return rank

            rank = lax.fori_loop(0, ngroups, eblk,
                                 jnp.zeros((L,), jnp.int32))
            sel = rank < K
            plsc.store_scatter(sc_v, [rank], vq, mask=sel)
            c_idx = iq & (CPAD - 1)
            q_idx = iq >> 7
            plsc.store_scatter(lb_v, [rank], c_idx, mask=sel)
            q4 = q_idx * 4
            cx = plsc.load_gather(box_v, [q4], mask=sel)
            cy = plsc.load_gather(box_v, [q4 + 1], mask=sel)
            w = plsc.load_gather(box_v, [q4 + 2], mask=sel)
            h = plsc.load_gather(box_v, [q4 + 3], mask=sel)
            r4 = rank * 4
            plsc.store_scatter(bx_v, [r4], (cx - 0.5 * w) * w_vec, mask=sel)
            plsc.store_scatter(bx_v, [r4 + 1], (cy - 0.5 * h) * h_vec,
                               mask=sel)
            plsc.store_scatter(bx_v, [r4 + 2], (cx + 0.5 * w) * w_vec,
                               mask=sel)
            plsc.store_scatter(bx_v, [r4 + 3], (cy + 0.5 * h) * h_vec,
                               mask=sel)
            return 0

        lax.fori_loop(0, ngroups, rank_group, 0)

        pltpu.sync_copy(sc_v, scores_hbm.at[pl.ds(b * KPAD, KPAD)])
        pltpu.sync_copy(lb_v, labels_hbm.at[pl.ds(b * KPAD, KPAD)])
        pltpu.sync_copy(bx_v, boxout_hbm.at[pl.ds(b * 4 * KPAD, 4 * KPAD)])


_sc_stage = pl.kernel(
    _sc_body,
    out_type=[
        jax.ShapeDtypeStruct((B * KPAD,), jnp.float32),
        jax.ShapeDtypeStruct((B * KPAD,), jnp.int32),
        jax.ShapeDtypeStruct((B * 4 * KPAD,), jnp.float32),
    ],
    mesh=plsc.VectorSubcoreMesh(core_axis_name="c", subcore_axis_name="s",
                                num_cores=NC, num_subcores=NS),
    compiler_params=pltpu.CompilerParams(needs_layout_passes=False),
    scratch_types=[
        pltpu.VMEM((CROWS, CPAD), jnp.float32),
        pltpu.VMEM((CANDBUF,), jnp.float32),
        pltpu.VMEM((CANDBUF,), jnp.int32),
        pltpu.VMEM((Q * 4,), jnp.float32),
        pltpu.VMEM((3 * L,), jnp.float32),
        pltpu.VMEM((KPAD,), jnp.float32),
        pltpu.VMEM((KPAD,), jnp.int32),
        pltpu.VMEM((4 * KPAD,), jnp.float32),
    ],
)


def kernel(pred_logits, pred_boxes, target_sizes, positive_map):
    prob, tvals = _tc_stage(pred_logits, positive_map)
    tvals = tvals.reshape(B, L)
    img_h = jnp.broadcast_to(target_sizes[:, 0:1], (B, L))
    img_w = jnp.broadcast_to(target_sizes[:, 1:2], (B, L))
    aux = jnp.concatenate([tvals, img_w, img_h], axis=1).reshape(-1)
    boxes_flat = pred_boxes.reshape(-1)
    scores_p, labels_p, boxes_p = _sc_stage(prob, aux, boxes_flat)
    scores = scores_p.reshape(B, KPAD)[:, :K]
    labels = labels_p.reshape(B, KPAD)[:, :K]
    boxes = boxes_p.reshape(B, KPAD, 4)[:, :K, :]
    return scores, labels, boxes
